# baseline (speedup 1.0000x reference)
import jax
import jax.numpy as jnp
from jax.experimental import pallas as pl
from jax.experimental.pallas import tpu as pltpu

N_SRT = 32768
SRT_H = 1024
COM_H = 768
B = 8
K = 2
TN = 2048

_NEG = float("-inf")
_BIGI = 2**31 - 1


def _score_topk_kernel(comments_ref, W_c_ref, b_c_ref, srt_ref, W_s_ref,
                       b_s_ref, idx_out_ref, tc_ref, v1, v2, i1, i2):
    step = pl.program_id(0)
    nsteps = pl.num_programs(0)

    @pl.when(step == 0)
    def _():
        tc = jax.nn.relu(
            jax.lax.dot_general(comments_ref[...], W_c_ref[...],
                                (((1,), (1,)), ((), ())),
                                preferred_element_type=jnp.float32)
            + b_c_ref[...])
        tc_ref[...] = tc

    t = jax.nn.relu(
        jax.lax.dot_general(srt_ref[...], W_s_ref[...],
                            (((1,), (1,)), ((), ())),
                            preferred_element_type=jnp.float32)
        + b_s_ref[...])
    s = jax.lax.dot_general(tc_ref[...], t, (((1,), (1,)), ((), ())),
                            preferred_element_type=jnp.float32)

    col = jax.lax.broadcasted_iota(jnp.int32, s.shape, 1) + step * TN
    bm1 = jnp.max(s, axis=1, keepdims=True)
    bi1 = jnp.min(jnp.where(s == bm1, col, _BIGI), axis=1, keepdims=True)
    masked = jnp.where(col == bi1, _NEG, s)
    bm2 = jnp.max(masked, axis=1, keepdims=True)
    bi2 = jnp.min(jnp.where(masked == bm2, col, _BIGI), axis=1, keepdims=True)

    @pl.when(step == 0)
    def _():
        v1[...] = jnp.broadcast_to(bm1, v1.shape)
        i1[...] = jnp.broadcast_to(bi1, i1.shape)
        v2[...] = jnp.broadcast_to(bm2, v2.shape)
        i2[...] = jnp.broadcast_to(bi2, i2.shape)

    @pl.when(step != 0)
    def _():
        a1 = v1[:, 0:1]
        a2 = v2[:, 0:1]
        ai1 = i1[:, 0:1]
        ai2 = i2[:, 0:1]
        cond1 = a1 >= bm1
        nv1 = jnp.where(cond1, a1, bm1)
        ni1 = jnp.where(cond1, ai1, bi1)
        c_a = a2 >= bm1
        c_b = a1 >= bm2
        nv2 = jnp.where(cond1, jnp.where(c_a, a2, bm1),
                        jnp.where(c_b, a1, bm2))
        ni2 = jnp.where(cond1, jnp.where(c_a, ai2, bi1),
                        jnp.where(c_b, ai1, bi2))
        v1[...] = jnp.broadcast_to(nv1, v1.shape)
        i1[...] = jnp.broadcast_to(ni1, i1.shape)
        v2[...] = jnp.broadcast_to(nv2, v2.shape)
        i2[...] = jnp.broadcast_to(ni2, i2.shape)

    @pl.when(step == nsteps - 1)
    def _():
        idx_out_ref[...] = jnp.concatenate([i1[...], i2[...]], axis=1)


def _gather_transform_kernel(idx_ref, srt_row_ref, W_s_ref, b_s_ref, out_ref):
    del idx_ref
    out_ref[...] = jax.nn.relu(
        jax.lax.dot_general(srt_row_ref[0], W_s_ref[...],
                            (((1,), (1,)), ((), ())),
                            preferred_element_type=jnp.float32)
        + b_s_ref[...])[None]


def kernel(hidden_states_srt, hidden_states_comments, W_c, b_c, W_s, b_s):
    b_c2 = b_c.reshape(1, SRT_H)
    b_s2 = b_s.reshape(1, SRT_H)

    idx_out = pl.pallas_call(
        _score_topk_kernel,
        grid=(N_SRT // TN,),
        in_specs=[
            pl.BlockSpec((B, COM_H), lambda i: (0, 0)),
            pl.BlockSpec((SRT_H, COM_H), lambda i: (0, 0)),
            pl.BlockSpec((1, SRT_H), lambda i: (0, 0)),
            pl.BlockSpec((TN, SRT_H), lambda i: (i, 0)),
            pl.BlockSpec((SRT_H, SRT_H), lambda i: (0, 0)),
            pl.BlockSpec((1, SRT_H), lambda i: (0, 0)),
        ],
        out_specs=pl.BlockSpec((B, 256), lambda i: (0, 0)),
        out_shape=jax.ShapeDtypeStruct((B, 256), jnp.int32),
        scratch_shapes=[
            pltpu.VMEM((B, SRT_H), jnp.float32),
            pltpu.VMEM((B, 128), jnp.float32),
            pltpu.VMEM((B, 128), jnp.float32),
            pltpu.VMEM((B, 128), jnp.int32),
            pltpu.VMEM((B, 128), jnp.int32),
        ],
    )(hidden_states_comments, W_c, b_c2, hidden_states_srt, W_s, b_s2)

    top_idx = jnp.stack([idx_out[:, 0], idx_out[:, 128]], axis=1)
    idx_flat = top_idx.reshape(B * K)

    srt3 = hidden_states_srt.reshape(N_SRT, 1, SRT_H)
    rows = pl.pallas_call(
        _gather_transform_kernel,
        grid_spec=pltpu.PrefetchScalarGridSpec(
            num_scalar_prefetch=1,
            grid=(B * K,),
            in_specs=[
                pl.BlockSpec((1, 1, SRT_H), lambda i, idx_ref: (idx_ref[i], 0, 0)),
                pl.BlockSpec((SRT_H, SRT_H), lambda i, idx_ref: (0, 0)),
                pl.BlockSpec((1, SRT_H), lambda i, idx_ref: (0, 0)),
            ],
            out_specs=pl.BlockSpec((1, 1, SRT_H), lambda i, idx_ref: (i, 0, 0)),
        ),
        out_shape=jax.ShapeDtypeStruct((B * K, 1, SRT_H), jnp.float32),
    )(idx_flat, srt3, W_s, b_s2)

    return rows.reshape(B, K, SRT_H)


# speedup vs baseline: 1.3042x; 1.3042x over previous
import jax
import jax.numpy as jnp
from jax.experimental import pallas as pl
from jax.experimental.pallas import tpu as pltpu

N_SRT = 32768
SRT_H = 1024
COM_H = 768
B = 8
K = 2
TN = 2048
NB = N_SRT // TN
M = 8

_NEG = float("-inf")
_BIGI = 2**31 - 1


def _phase1_kernel(comments_ref, W_c_ref, b_c_ref, srt_ref, W_s_ref, b_s_ref,
                   idx_out_ref, tbuf_ref, tcb_ref, wsb_ref, bsb_ref, *slots):
    cv = slots[:M]
    ci = slots[M:]
    step = pl.program_id(0)

    @pl.when(step == 0)
    def _():
        tc = jax.nn.relu(
            jax.lax.dot_general(comments_ref[...], W_c_ref[...],
                                (((1,), (1,)), ((), ())),
                                preferred_element_type=jnp.float32)
            + b_c_ref[...])
        tcb_ref[...] = tc.astype(jnp.bfloat16)
        wsb_ref[...] = W_s_ref[...].astype(jnp.bfloat16)
        bsb_ref[...] = b_s_ref[...].astype(jnp.bfloat16)

    @pl.when(step >= 1)
    def _():
        s = jax.lax.dot_general(tcb_ref[...], tbuf_ref[...],
                                (((1,), (1,)), ((), ())),
                                preferred_element_type=jnp.float32)
        col = jax.lax.broadcasted_iota(jnp.int32, s.shape, 1) + (step - 1) * TN
        bvals = []
        bidxs = []
        for _ in range(M):
            m = jnp.max(s, axis=1, keepdims=True)
            i = jnp.min(jnp.where(s == m, col, _BIGI), axis=1, keepdims=True)
            bvals.append(m)
            bidxs.append(i)
            s = jnp.where(col == i, _NEG, s)

        @pl.when(step == 1)
        def _():
            for k in range(M):
                cv[k][...] = jnp.broadcast_to(bvals[k], cv[k].shape)
                ci[k][...] = jnp.broadcast_to(bidxs[k], ci[k].shape)

        @pl.when(step != 1)
        def _():
            cand_v = jnp.concatenate(
                [cv[k][:, 0:1] for k in range(M)] + bvals, axis=1)
            cand_i = jnp.concatenate(
                [ci[k][:, 0:1] for k in range(M)] + bidxs, axis=1)
            for k in range(M):
                m = jnp.max(cand_v, axis=1, keepdims=True)
                gi = jnp.min(jnp.where(cand_v == m, cand_i, _BIGI),
                             axis=1, keepdims=True)
                cv[k][...] = jnp.broadcast_to(m, cv[k].shape)
                ci[k][...] = jnp.broadcast_to(gi, ci[k].shape)
                cand_v = jnp.where((cand_v == m) & (cand_i == gi), _NEG,
                                   cand_v)

    @pl.when(step < NB)
    def _():
        sb = srt_ref[...].astype(jnp.bfloat16)
        mm = jax.lax.dot_general(sb, wsb_ref[...], (((1,), (1,)), ((), ())),
                                 preferred_element_type=jnp.float32)
        xb = mm.astype(jnp.bfloat16)
        tbuf_ref[...] = jnp.maximum(xb + bsb_ref[...], jnp.bfloat16(0.0))

    @pl.when(step == NB)
    def _():
        idx_out_ref[...] = jnp.concatenate([ci[k][...] for k in range(M)],
                                           axis=1)


def _phase2_kernel(idx_sref, srt_ref, idxv_ref, com_ref, W_c_ref, b_c_ref,
                   W_s_ref, b_s_ref, out_ref, rows_ref, sem):
    copies = []
    for k in range(B * M):
        c = pltpu.make_async_copy(
            srt_ref.at[pl.ds(idx_sref[k], 1), :],
            rows_ref.at[pl.ds(k, 1), :],
            sem)
        c.start()
        copies.append(c)
    for c in copies:
        c.wait()

    t64 = jax.nn.relu(
        jax.lax.dot_general(rows_ref[...], W_s_ref[...],
                            (((1,), (1,)), ((), ())),
                            preferred_element_type=jnp.float32)
        + b_s_ref[...])
    tc16 = jax.nn.relu(
        jax.lax.dot_general(com_ref[...], W_c_ref[...],
                            (((1,), (1,)), ((), ())),
                            preferred_element_type=jnp.float32)
        + b_c_ref[...])
    s16 = jax.lax.dot_general(tc16, t64, (((1,), (1,)), ((), ())),
                              preferred_element_type=jnp.float32)

    jio = jax.lax.broadcasted_iota(jnp.int32, s16.shape, 1)
    rio = jax.lax.broadcasted_iota(jnp.int32, s16.shape, 0)
    valid = (jio // M) == (rio // K)
    gidx = jnp.broadcast_to(idxv_ref[...], s16.shape)
    sm = jnp.where(valid, s16, _NEG)
    m1 = jnp.max(sm, axis=1, keepdims=True)
    g1 = jnp.min(jnp.where(sm == m1, gidx, _BIGI), axis=1, keepdims=True)
    best1 = (sm == m1) & (gidx == g1)
    sm2 = jnp.where(best1, _NEG, sm)
    m2 = jnp.max(sm2, axis=1, keepdims=True)
    g2 = jnp.min(jnp.where(sm2 == m2, gidx, _BIGI), axis=1, keepdims=True)
    best2 = (sm2 == m2) & (gidx == g2)
    is2 = (rio % K) == 1
    P = jnp.where(is2, best2.astype(jnp.float32), best1.astype(jnp.float32))
    acc = P[:, 0:1] * t64[0:1, :]
    for j in range(1, B * M):
        acc = acc + P[:, j:j + 1] * t64[j:j + 1, :]
    out_ref[...] = acc


def kernel(hidden_states_srt, hidden_states_comments, W_c, b_c, W_s, b_s):
    b_c2 = b_c.reshape(1, SRT_H)
    b_s2 = b_s.reshape(1, SRT_H)

    idx_out = pl.pallas_call(
        _phase1_kernel,
        grid=(NB + 1,),
        in_specs=[
            pl.BlockSpec((B, COM_H), lambda i: (0, 0)),
            pl.BlockSpec((SRT_H, COM_H), lambda i: (0, 0)),
            pl.BlockSpec((1, SRT_H), lambda i: (0, 0)),
            pl.BlockSpec((TN, SRT_H), lambda i: (jnp.minimum(i, NB - 1), 0)),
            pl.BlockSpec((SRT_H, SRT_H), lambda i: (0, 0)),
            pl.BlockSpec((1, SRT_H), lambda i: (0, 0)),
        ],
        out_specs=pl.BlockSpec((B, M * 128), lambda i: (0, 0)),
        out_shape=jax.ShapeDtypeStruct((B, M * 128), jnp.int32),
        scratch_shapes=(
            [pltpu.VMEM((TN, SRT_H), jnp.bfloat16),
             pltpu.VMEM((B, SRT_H), jnp.bfloat16),
             pltpu.VMEM((SRT_H, SRT_H), jnp.bfloat16),
             pltpu.VMEM((1, SRT_H), jnp.bfloat16)]
            + [pltpu.VMEM((B, 128), jnp.float32) for _ in range(M)]
            + [pltpu.VMEM((B, 128), jnp.int32) for _ in range(M)]
        ),
    )(hidden_states_comments, W_c, b_c2, hidden_states_srt, W_s, b_s2)

    idx_flat = idx_out[:, ::128].reshape(B * M)
    idx_vec = idx_flat.reshape(1, B * M)
    com16 = jnp.repeat(hidden_states_comments, K, axis=0)

    rows = pl.pallas_call(
        _phase2_kernel,
        grid_spec=pltpu.PrefetchScalarGridSpec(
            num_scalar_prefetch=1,
            grid=(1,),
            in_specs=[
                pl.BlockSpec(memory_space=pl.ANY),
                pl.BlockSpec((1, B * M), lambda i, idx_ref: (0, 0)),
                pl.BlockSpec((B * K, COM_H), lambda i, idx_ref: (0, 0)),
                pl.BlockSpec((SRT_H, COM_H), lambda i, idx_ref: (0, 0)),
                pl.BlockSpec((1, SRT_H), lambda i, idx_ref: (0, 0)),
                pl.BlockSpec((SRT_H, SRT_H), lambda i, idx_ref: (0, 0)),
                pl.BlockSpec((1, SRT_H), lambda i, idx_ref: (0, 0)),
            ],
            out_specs=pl.BlockSpec((B * K, SRT_H), lambda i, idx_ref: (0, 0)),
            scratch_shapes=[
                pltpu.VMEM((B * M, SRT_H), jnp.float32),
                pltpu.SemaphoreType.DMA,
            ],
        ),
        out_shape=jax.ShapeDtypeStruct((B * K, SRT_H), jnp.float32),
    )(idx_flat, hidden_states_srt, idx_vec, com16, W_c, b_c2, W_s, b_s2)

    return rows.reshape(B, K, SRT_H)


# speedup vs baseline: 1.8041x; 1.3834x over previous
import jax
import jax.numpy as jnp
from jax.experimental import pallas as pl
from jax.experimental.pallas import tpu as pltpu

N_SRT = 32768
SRT_H = 1024
COM_H = 768
B = 8
K = 2
TN = 2048
NB = N_SRT // TN
M = 4
NC = NB * M

_NEG = float("-inf")
_BIGI = 2**31 - 1


def _phase1_kernel(comments_ref, W_c_ref, b_c_ref, srt_ref, W_s_ref, b_s_ref,
                   cand_ref, tbuf_ref, tcb_ref, wsb_ref, bsb_ref):
    step = pl.program_id(0)

    @pl.when(step == 0)
    def _():
        tc = jax.nn.relu(
            jax.lax.dot_general(comments_ref[...], W_c_ref[...],
                                (((1,), (1,)), ((), ())),
                                preferred_element_type=jnp.float32)
            + b_c_ref[...])
        tcb_ref[...] = tc.astype(jnp.bfloat16)
        wsb_ref[...] = W_s_ref[...].astype(jnp.bfloat16)
        bsb_ref[...] = b_s_ref[...].astype(jnp.bfloat16)

    s = jax.lax.dot_general(tcb_ref[...], tbuf_ref[...],
                            (((1,), (1,)), ((), ())),
                            preferred_element_type=jnp.float32)
    col = jax.lax.broadcasted_iota(jnp.int32, s.shape, 1) + (step - 1) * TN
    pieces = []
    for _ in range(M):
        m = jnp.max(s, axis=1, keepdims=True)
        i = jnp.min(jnp.where(s == m, col, _BIGI), axis=1, keepdims=True)
        pieces.append(jnp.broadcast_to(i, (B, 128)))
        s = jnp.where(col == i, _NEG, s)
    cand_ref[...] = jnp.concatenate(pieces, axis=1)[None]

    sb = srt_ref[...].astype(jnp.bfloat16)
    mm = jax.lax.dot_general(sb, wsb_ref[...], (((1,), (1,)), ((), ())),
                             preferred_element_type=jnp.float32)
    xb = mm.astype(jnp.bfloat16)
    tbuf_ref[...] = jnp.maximum(xb + bsb_ref[...], jnp.bfloat16(0.0))


def _phase2_kernel(idx_sref, srt_ref, idxv_ref, com_ref, W_c_ref, b_c_ref,
                   W_s_ref, b_s_ref, out_ref, rows_ref, sem):
    copies = []
    for k in range(B * NC):
        c = pltpu.make_async_copy(
            srt_ref.at[pl.ds(idx_sref[k], 1), :],
            rows_ref.at[pl.ds(k, 1), :],
            sem)
        c.start()
        copies.append(c)
    for c in copies:
        c.wait()

    t = jax.nn.relu(
        jax.lax.dot_general(rows_ref[...], W_s_ref[...],
                            (((1,), (1,)), ((), ())),
                            preferred_element_type=jnp.float32)
        + b_s_ref[...])
    tc = jax.nn.relu(
        jax.lax.dot_general(com_ref[...], W_c_ref[...],
                            (((1,), (1,)), ((), ())),
                            preferred_element_type=jnp.float32)
        + b_c_ref[...])
    s = jax.lax.dot_general(tc, t, (((1,), (1,)), ((), ())),
                            preferred_element_type=jnp.float32)

    jio = jax.lax.broadcasted_iota(jnp.int32, s.shape, 1)
    rio = jax.lax.broadcasted_iota(jnp.int32, s.shape, 0)
    valid = (jio // NC) == rio
    gidx = jnp.broadcast_to(idxv_ref[...], s.shape)
    sm = jnp.where(valid, s, _NEG)
    m1 = jnp.max(sm, axis=1, keepdims=True)
    g1 = jnp.min(jnp.where(sm == m1, gidx, _BIGI), axis=1, keepdims=True)
    best1 = (sm == m1) & (gidx == g1)
    sm2 = jnp.where(best1, _NEG, sm)
    m2 = jnp.max(sm2, axis=1, keepdims=True)
    g2 = jnp.min(jnp.where(sm2 == m2, gidx, _BIGI), axis=1, keepdims=True)
    out_ref[...] = jnp.concatenate(
        [jnp.broadcast_to(g1, (B, 128)), jnp.broadcast_to(g2, (B, 128))],
        axis=1)


def _phase3_kernel(idx_sref, srt_ref, W_s_ref, b_s_ref, out_ref, rows_ref,
                   sem):
    copies = []
    for k in range(B * K):
        c = pltpu.make_async_copy(
            srt_ref.at[pl.ds(idx_sref[k], 1), :],
            rows_ref.at[pl.ds(k, 1), :],
            sem)
        c.start()
        copies.append(c)
    for c in copies:
        c.wait()
    out_ref[...] = jax.nn.relu(
        jax.lax.dot_general(rows_ref[...], W_s_ref[...],
                            (((1,), (1,)), ((), ())),
                            preferred_element_type=jnp.float32)
        + b_s_ref[...])


def kernel(hidden_states_srt, hidden_states_comments, W_c, b_c, W_s, b_s):
    b_c2 = b_c.reshape(1, SRT_H)
    b_s2 = b_s.reshape(1, SRT_H)

    cand = pl.pallas_call(
        _phase1_kernel,
        grid=(NB + 1,),
        in_specs=[
            pl.BlockSpec((B, COM_H), lambda i: (0, 0)),
            pl.BlockSpec((SRT_H, COM_H), lambda i: (0, 0)),
            pl.BlockSpec((1, SRT_H), lambda i: (0, 0)),
            pl.BlockSpec((TN, SRT_H), lambda i: (jnp.minimum(i, NB - 1), 0)),
            pl.BlockSpec((SRT_H, SRT_H), lambda i: (0, 0)),
            pl.BlockSpec((1, SRT_H), lambda i: (0, 0)),
        ],
        out_specs=pl.BlockSpec((1, B, M * 128), lambda i: (i, 0, 0)),
        out_shape=jax.ShapeDtypeStruct((NB + 1, B, M * 128), jnp.int32),
        scratch_shapes=[
            pltpu.VMEM((TN, SRT_H), jnp.bfloat16),
            pltpu.VMEM((B, SRT_H), jnp.bfloat16),
            pltpu.VMEM((SRT_H, SRT_H), jnp.bfloat16),
            pltpu.VMEM((1, SRT_H), jnp.bfloat16),
        ],
    )(hidden_states_comments, W_c, b_c2, hidden_states_srt, W_s, b_s2)

    cand_idx = cand[1:, :, ::128]
    cand_flat = cand_idx.transpose(1, 0, 2).reshape(B * NC)
    cand_vec = cand_flat.reshape(1, B * NC)

    top2 = pl.pallas_call(
        _phase2_kernel,
        grid_spec=pltpu.PrefetchScalarGridSpec(
            num_scalar_prefetch=1,
            grid=(1,),
            in_specs=[
                pl.BlockSpec(memory_space=pl.ANY),
                pl.BlockSpec((1, B * NC), lambda i, idx_ref: (0, 0)),
                pl.BlockSpec((B, COM_H), lambda i, idx_ref: (0, 0)),
                pl.BlockSpec((SRT_H, COM_H), lambda i, idx_ref: (0, 0)),
                pl.BlockSpec((1, SRT_H), lambda i, idx_ref: (0, 0)),
                pl.BlockSpec((SRT_H, SRT_H), lambda i, idx_ref: (0, 0)),
                pl.BlockSpec((1, SRT_H), lambda i, idx_ref: (0, 0)),
            ],
            out_specs=pl.BlockSpec((B, 256), lambda i, idx_ref: (0, 0)),
            scratch_shapes=[
                pltpu.VMEM((B * NC, SRT_H), jnp.float32),
                pltpu.SemaphoreType.DMA,
            ],
        ),
        out_shape=jax.ShapeDtypeStruct((B, 256), jnp.int32),
    )(cand_flat, hidden_states_srt, cand_vec, hidden_states_comments, W_c,
      b_c2, W_s, b_s2)

    idx16 = top2[:, ::128].reshape(B * K)

    rows = pl.pallas_call(
        _phase3_kernel,
        grid_spec=pltpu.PrefetchScalarGridSpec(
            num_scalar_prefetch=1,
            grid=(1,),
            in_specs=[
                pl.BlockSpec(memory_space=pl.ANY),
                pl.BlockSpec((SRT_H, SRT_H), lambda i, idx_ref: (0, 0)),
                pl.BlockSpec((1, SRT_H), lambda i, idx_ref: (0, 0)),
            ],
            out_specs=pl.BlockSpec((B * K, SRT_H), lambda i, idx_ref: (0, 0)),
            scratch_shapes=[
                pltpu.VMEM((B * K, SRT_H), jnp.float32),
                pltpu.SemaphoreType.DMA,
            ],
        ),
        out_shape=jax.ShapeDtypeStruct((B * K, SRT_H), jnp.float32),
    )(idx16, hidden_states_srt, W_s, b_s2)

    return rows.reshape(B, K, SRT_H)
